# baseline (device time: 33101 ns/iter reference)
import jax
import jax.numpy as jnp
from jax import lax
from jax.experimental import pallas as pl
from jax.experimental.pallas import tpu as pltpu

N_DEV = 8
LOG_N = 3
BLK = 64


def kernel(x, Wq, K_ext, V_ext, Wo):
    B, Sq, Dm = x.shape
    _, Dq = Wq.shape
    _, Skv, Hq, Dh = K_ext.shape
    HL = Dq // Dh
    my = lax.axis_index("i")

    xb = x.reshape(B * Sq, Dm).astype(jnp.bfloat16)
    Wqb = Wq.astype(jnp.bfloat16)
    Wob = Wo.astype(jnp.bfloat16)
    K_loc = lax.dynamic_slice_in_dim(K_ext, my * HL, HL, axis=2).astype(jnp.bfloat16)
    V_loc = lax.dynamic_slice_in_dim(V_ext, my * HL, HL, axis=2).astype(jnp.bfloat16)

    def body(x_ref, wq_ref, k_ref, v_ref, wo_ref, out_ref,
             ctx_ref, cbuf_ref, send_sems, recv_sems):
        my_pos = lax.axis_index("i")

        barrier = pltpu.get_barrier_semaphore()
        for r in range(LOG_N):
            pl.semaphore_signal(
                barrier, inc=1,
                device_id=(my_pos ^ (1 << r),),
                device_id_type=pl.DeviceIdType.MESH,
            )
        pl.semaphore_wait(barrier, LOG_N)

        q = lax.dot_general(
            x_ref[...], wq_ref[...], (((1,), (0,)), ((), ())),
            preferred_element_type=jnp.float32,
        ).astype(jnp.bfloat16)

        i_idx = lax.broadcasted_iota(jnp.int32, (Sq, Skv), 0)
        j_idx = lax.broadcasted_iota(jnp.int32, (Sq, Skv), 1)
        mask = (j_idx // BLK) <= (i_idx // BLK)

        for b in range(B):
            for h in range(HL):
                q_bh = q[b * Sq:(b + 1) * Sq, h * Dh:(h + 1) * Dh]
                k_bh = k_ref[b, :, h, :]
                v_bh = v_ref[b, :, h, :]
                s = lax.dot_general(
                    q_bh, k_bh, (((1,), (1,)), ((), ())),
                    preferred_element_type=jnp.float32,
                ) * 0.125
                s = jnp.where(mask, s, -1e9)
                s = s - jnp.max(s, axis=-1, keepdims=True)
                w = jnp.exp(s)
                w = w / jnp.sum(w, axis=-1, keepdims=True)
                ctx = lax.dot_general(
                    w.astype(jnp.bfloat16), v_bh, (((1,), (0,)), ((), ())),
                    preferred_element_type=jnp.float32,
                )
                ctx_ref[b * Sq:(b + 1) * Sq, h * Dh:(h + 1) * Dh] = (
                    ctx.astype(jnp.bfloat16))

        out_ref[...] = lax.dot_general(
            ctx_ref[...], wo_ref[...], (((1,), (0,)), ((), ())),
            preferred_element_type=jnp.float32,
        )

        for r in range(LOG_N):
            partner = my_pos ^ (1 << r)
            rdma = pltpu.make_async_remote_copy(
                src_ref=out_ref,
                dst_ref=cbuf_ref.at[r],
                send_sem=send_sems.at[r],
                recv_sem=recv_sems.at[r],
                device_id=(partner,),
                device_id_type=pl.DeviceIdType.MESH,
            )
            rdma.start()
            rdma.wait()
            out_ref[...] = out_ref[...] + cbuf_ref[r]

    out2d = pl.pallas_call(
        body,
        out_shape=jax.ShapeDtypeStruct((B * Sq, Dm), jnp.float32),
        in_specs=[pl.BlockSpec(memory_space=pltpu.VMEM)] * 5,
        out_specs=pl.BlockSpec(memory_space=pltpu.VMEM),
        scratch_shapes=[
            pltpu.VMEM((B * Sq, HL * Dh), jnp.bfloat16),
            pltpu.VMEM((LOG_N, B * Sq, Dm), jnp.float32),
            pltpu.SemaphoreType.DMA((LOG_N,)),
            pltpu.SemaphoreType.DMA((LOG_N,)),
        ],
        compiler_params=pltpu.CompilerParams(collective_id=0),
    )(xb, Wqb, K_loc, V_loc, Wob)
    return out2d.reshape(B, Sq, Dm)


# device time: 22694 ns/iter; 1.4586x vs baseline; 1.4586x over previous
import jax
import jax.numpy as jnp
from jax import lax
from jax.experimental import pallas as pl
from jax.experimental.pallas import tpu as pltpu

N_DEV = 8
LOG_N = 3
BLK = 64


def kernel(x, Wq, K_ext, V_ext, Wo):
    B, Sq, Dm = x.shape
    _, Dq = Wq.shape
    _, Skv, Hq, Dh = K_ext.shape
    HL = Dq // Dh
    my = lax.axis_index("i")

    xb = x.reshape(B * Sq, Dm)
    K_loc = lax.dynamic_slice_in_dim(K_ext, my * HL, HL, axis=2)
    V_loc = lax.dynamic_slice_in_dim(V_ext, my * HL, HL, axis=2)

    def body(x_ref, wq_ref, k_ref, v_ref, wo_ref, out_ref,
             ctx_ref, acc_ref, cbuf_ref, send_sems, recv_sems):
        my_pos = lax.axis_index("i")

        xors = (1, 3, 4)

        barrier = pltpu.get_barrier_semaphore()
        for r in range(LOG_N):
            pl.semaphore_signal(
                barrier, inc=1,
                device_id=(my_pos ^ xors[r],),
                device_id_type=pl.DeviceIdType.MESH,
            )
        pl.semaphore_wait(barrier, LOG_N)

        q = lax.dot_general(
            x_ref[...].astype(jnp.bfloat16),
            wq_ref[...].astype(jnp.bfloat16), (((1,), (0,)), ((), ())),
            preferred_element_type=jnp.float32,
        ).astype(jnp.bfloat16)

        i_idx = lax.broadcasted_iota(jnp.int32, (Sq, Skv), 0)
        j_idx = lax.broadcasted_iota(jnp.int32, (Sq, Skv), 1)
        mask = (j_idx // BLK) <= (i_idx // BLK)

        for b in range(B):
            for h in range(HL):
                q_bh = q[b * Sq:(b + 1) * Sq, h * Dh:(h + 1) * Dh]
                k_bh = k_ref[b, :, h, :].astype(jnp.bfloat16)
                v_bh = v_ref[b, :, h, :].astype(jnp.bfloat16)
                s = lax.dot_general(
                    q_bh, k_bh, (((1,), (1,)), ((), ())),
                    preferred_element_type=jnp.float32,
                ) * 0.125
                s = jnp.where(mask, s, -1e9)
                s = s - jnp.max(s, axis=-1, keepdims=True)
                w = jnp.exp(s)
                w = w / jnp.sum(w, axis=-1, keepdims=True)
                ctx = lax.dot_general(
                    w.astype(jnp.bfloat16), v_bh, (((1,), (0,)), ((), ())),
                    preferred_element_type=jnp.float32,
                )
                ctx_ref[b * Sq:(b + 1) * Sq, h * Dh:(h + 1) * Dh] = (
                    ctx.astype(jnp.bfloat16))

        acc_ref[...] = lax.dot_general(
            ctx_ref[...], wo_ref[...].astype(jnp.bfloat16),
            (((1,), (0,)), ((), ())),
            preferred_element_type=jnp.float32,
        ).astype(jnp.bfloat16)

        for r in range(LOG_N):
            partner = my_pos ^ xors[r]
            rdma = pltpu.make_async_remote_copy(
                src_ref=acc_ref,
                dst_ref=cbuf_ref.at[r],
                send_sem=send_sems.at[r],
                recv_sem=recv_sems.at[r],
                device_id=(partner,),
                device_id_type=pl.DeviceIdType.MESH,
            )
            rdma.start()
            rdma.wait()
            acc_ref[...] = acc_ref[...] + cbuf_ref[r]

        out_ref[...] = acc_ref[...].astype(jnp.float32)

    out2d = pl.pallas_call(
        body,
        out_shape=jax.ShapeDtypeStruct((B * Sq, Dm), jnp.float32),
        in_specs=[pl.BlockSpec(memory_space=pltpu.VMEM)] * 5,
        out_specs=pl.BlockSpec(memory_space=pltpu.VMEM),
        scratch_shapes=[
            pltpu.VMEM((B * Sq, HL * Dh), jnp.bfloat16),
            pltpu.VMEM((B * Sq, Dm), jnp.bfloat16),
            pltpu.VMEM((LOG_N, B * Sq, Dm), jnp.bfloat16),
            pltpu.SemaphoreType.DMA((LOG_N,)),
            pltpu.SemaphoreType.DMA((LOG_N,)),
        ],
        compiler_params=pltpu.CompilerParams(collective_id=0),
    )(xb, Wq, K_loc, V_loc, Wo)
    return out2d.reshape(B, Sq, Dm)


# device time: 18867 ns/iter; 1.7544x vs baseline; 1.2028x over previous
import jax
import jax.numpy as jnp
from jax import lax
from jax.experimental import pallas as pl
from jax.experimental.pallas import tpu as pltpu

N_DEV = 8
LOG_N = 3
BLK = 64


def kernel(x, Wq, K_ext, V_ext, Wo):
    B, Sq, Dm = x.shape
    _, Dq = Wq.shape
    _, Skv, Hq, Dh = K_ext.shape
    HL = Dq // Dh
    my = lax.axis_index("i")

    xb = x.reshape(B * Sq, Dm)
    K_loc = lax.dynamic_slice_in_dim(K_ext, my * HL, HL, axis=2)
    V_loc = lax.dynamic_slice_in_dim(V_ext, my * HL, HL, axis=2)

    def body(x_ref, wq_ref, k_ref, v_ref, wo_ref, out_ref,
             ctx_ref, acc_ref, cbuf_ref, send_sems, recv_sems):
        my_pos = lax.axis_index("i")

        xors = (1, 3, 4)

        barrier = pltpu.get_barrier_semaphore()
        for r in range(LOG_N):
            pl.semaphore_signal(
                barrier, inc=1,
                device_id=(my_pos ^ xors[r],),
                device_id_type=pl.DeviceIdType.MESH,
            )
        pl.semaphore_wait(barrier, LOG_N)

        q = lax.dot_general(
            x_ref[...].astype(jnp.bfloat16),
            wq_ref[...].astype(jnp.bfloat16), (((1,), (0,)), ((), ())),
            preferred_element_type=jnp.float32,
        ).astype(jnp.bfloat16)

        i_idx = lax.broadcasted_iota(jnp.int32, (Sq, Skv), 0)
        j_idx = lax.broadcasted_iota(jnp.int32, (Sq, Skv), 1)
        mask = (j_idx // BLK) <= (i_idx // BLK)

        def make_rdma(b, r):
            rows = pl.ds(b * Sq, Sq)
            return pltpu.make_async_remote_copy(
                src_ref=acc_ref.at[rows],
                dst_ref=cbuf_ref.at[r, rows],
                send_sem=send_sems.at[b, r],
                recv_sem=recv_sems.at[b, r],
                device_id=(my_pos ^ xors[r],),
                device_id_type=pl.DeviceIdType.MESH,
            )

        rdmas = {}
        for b in range(B):
            for h in range(HL):
                q_bh = q[b * Sq:(b + 1) * Sq, h * Dh:(h + 1) * Dh]
                k_bh = k_ref[b, :, h, :].astype(jnp.bfloat16)
                v_bh = v_ref[b, :, h, :].astype(jnp.bfloat16)
                s = lax.dot_general(
                    q_bh, k_bh, (((1,), (1,)), ((), ())),
                    preferred_element_type=jnp.float32,
                ) * 0.125
                s = jnp.where(mask, s, -1e9)
                s = s - jnp.max(s, axis=-1, keepdims=True)
                w = jnp.exp(s)
                w = w / jnp.sum(w, axis=-1, keepdims=True)
                ctx = lax.dot_general(
                    w.astype(jnp.bfloat16), v_bh, (((1,), (0,)), ((), ())),
                    preferred_element_type=jnp.float32,
                )
                ctx_ref[b * Sq:(b + 1) * Sq, h * Dh:(h + 1) * Dh] = (
                    ctx.astype(jnp.bfloat16))
            acc_ref[b * Sq:(b + 1) * Sq, :] = lax.dot_general(
                ctx_ref[b * Sq:(b + 1) * Sq, :],
                wo_ref[...].astype(jnp.bfloat16),
                (((1,), (0,)), ((), ())),
                preferred_element_type=jnp.float32,
            ).astype(jnp.bfloat16)
            rdmas[(b, 0)] = make_rdma(b, 0)
            rdmas[(b, 0)].start()

        for r in range(LOG_N):
            for b in range(B):
                rdmas[(b, r)].wait()
                rows = slice(b * Sq, (b + 1) * Sq)
                acc_ref[rows, :] = acc_ref[rows, :] + cbuf_ref[r, rows, :]
                if r + 1 < LOG_N:
                    rdmas[(b, r + 1)] = make_rdma(b, r + 1)
                    rdmas[(b, r + 1)].start()

        out_ref[...] = acc_ref[...].astype(jnp.float32)

    out2d = pl.pallas_call(
        body,
        out_shape=jax.ShapeDtypeStruct((B * Sq, Dm), jnp.float32),
        in_specs=[pl.BlockSpec(memory_space=pltpu.VMEM)] * 5,
        out_specs=pl.BlockSpec(memory_space=pltpu.VMEM),
        scratch_shapes=[
            pltpu.VMEM((B * Sq, HL * Dh), jnp.bfloat16),
            pltpu.VMEM((B * Sq, Dm), jnp.bfloat16),
            pltpu.VMEM((LOG_N, B * Sq, Dm), jnp.bfloat16),
            pltpu.SemaphoreType.DMA((B, LOG_N)),
            pltpu.SemaphoreType.DMA((B, LOG_N)),
        ],
        compiler_params=pltpu.CompilerParams(collective_id=0),
    )(xb, Wq, K_loc, V_loc, Wo)
    return out2d.reshape(B, Sq, Dm)


# device time: 18181 ns/iter; 1.8206x vs baseline; 1.0377x over previous
import jax
import jax.numpy as jnp
from jax import lax
from jax.experimental import pallas as pl
from jax.experimental.pallas import tpu as pltpu

N_DEV = 8
LOG_N = 3
BLK = 64


def kernel(x, Wq, K_ext, V_ext, Wo):
    B, Sq, Dm = x.shape
    _, Dq = Wq.shape
    _, Skv, Hq, Dh = K_ext.shape
    HL = Dq // Dh
    my = lax.axis_index("i")

    xb = x.reshape(B * Sq, Dm)
    K_loc = lax.dynamic_slice_in_dim(K_ext, my * HL, HL, axis=2)
    V_loc = lax.dynamic_slice_in_dim(V_ext, my * HL, HL, axis=2)

    def body(x_ref, wq_ref, k_ref, v_ref, wo_ref, out_ref,
             ctx_ref, acc_ref, cbuf_ref, send_sems, recv_sems):
        my_pos = lax.axis_index("i")

        xors = (1, 3, 4)

        barrier = pltpu.get_barrier_semaphore()
        for r in range(LOG_N):
            pl.semaphore_signal(
                barrier, inc=1,
                device_id=(my_pos ^ xors[r],),
                device_id_type=pl.DeviceIdType.MESH,
            )
        pl.semaphore_wait(barrier, LOG_N)

        q = (lax.dot_general(
            x_ref[...].astype(jnp.bfloat16),
            wq_ref[...].astype(jnp.bfloat16), (((1,), (0,)), ((), ())),
            preferred_element_type=jnp.float32,
        ) * 0.125).astype(jnp.bfloat16)

        i_idx = lax.broadcasted_iota(jnp.int32, (Sq, Skv), 0)
        j_idx = lax.broadcasted_iota(jnp.int32, (Sq, Skv), 1)
        bias = jnp.where((j_idx // BLK) <= (i_idx // BLK), 0.0, -1e9)

        def make_rdma(b, r):
            rows = pl.ds(b * Sq, Sq)
            return pltpu.make_async_remote_copy(
                src_ref=acc_ref.at[rows],
                dst_ref=cbuf_ref.at[r, rows],
                send_sem=send_sems.at[b, r],
                recv_sem=recv_sems.at[b, r],
                device_id=(my_pos ^ xors[r],),
                device_id_type=pl.DeviceIdType.MESH,
            )

        rdmas = {}
        for b in range(B):
            for h in range(HL):
                q_bh = q[b * Sq:(b + 1) * Sq, h * Dh:(h + 1) * Dh]
                k_bh = k_ref[b, :, h, :].astype(jnp.bfloat16)
                v_bh = v_ref[b, :, h, :].astype(jnp.bfloat16)
                s = lax.dot_general(
                    q_bh, k_bh, (((1,), (1,)), ((), ())),
                    preferred_element_type=jnp.float32,
                ) + bias
                w = jnp.exp(s)
                recip = 1.0 / jnp.sum(w, axis=-1, keepdims=True)
                ctx = lax.dot_general(
                    w.astype(jnp.bfloat16), v_bh, (((1,), (0,)), ((), ())),
                    preferred_element_type=jnp.float32,
                ) * recip
                ctx_ref[b * Sq:(b + 1) * Sq, h * Dh:(h + 1) * Dh] = (
                    ctx.astype(jnp.bfloat16))
            acc_ref[b * Sq:(b + 1) * Sq, :] = lax.dot_general(
                ctx_ref[b * Sq:(b + 1) * Sq, :],
                wo_ref[...].astype(jnp.bfloat16),
                (((1,), (0,)), ((), ())),
                preferred_element_type=jnp.float32,
            ).astype(jnp.bfloat16)
            rdmas[(b, 0)] = make_rdma(b, 0)
            rdmas[(b, 0)].start()

        for r in range(LOG_N):
            for b in range(B):
                rdmas[(b, r)].wait()
                rows = slice(b * Sq, (b + 1) * Sq)
                acc_ref[rows, :] = acc_ref[rows, :] + cbuf_ref[r, rows, :]
                if r + 1 < LOG_N:
                    rdmas[(b, r + 1)] = make_rdma(b, r + 1)
                    rdmas[(b, r + 1)].start()

        out_ref[...] = acc_ref[...].astype(jnp.float32)

    out2d = pl.pallas_call(
        body,
        out_shape=jax.ShapeDtypeStruct((B * Sq, Dm), jnp.float32),
        in_specs=[pl.BlockSpec(memory_space=pltpu.VMEM)] * 5,
        out_specs=pl.BlockSpec(memory_space=pltpu.VMEM),
        scratch_shapes=[
            pltpu.VMEM((B * Sq, HL * Dh), jnp.bfloat16),
            pltpu.VMEM((B * Sq, Dm), jnp.bfloat16),
            pltpu.VMEM((LOG_N, B * Sq, Dm), jnp.bfloat16),
            pltpu.SemaphoreType.DMA((B, LOG_N)),
            pltpu.SemaphoreType.DMA((B, LOG_N)),
        ],
        compiler_params=pltpu.CompilerParams(collective_id=0),
    )(xb, Wq, K_loc, V_loc, Wo)
    return out2d.reshape(B, Sq, Dm)
